# baseline (device time: 274350 ns/iter reference)
import jax
import jax.numpy as jnp
from jax import lax
from jax.experimental import pallas as pl
from jax.experimental.pallas import tpu as pltpu

N_DEV = 16
N_Z = 4
N_T = 4

_ZTAB = [
    0, 1, 2, 3,
    1, 0, 2, 3,
    2, 1, 3, 0,
    3, 2, 1, 0,
]


def kernel(A, B):
    m_per, k = A.shape
    m_half = m_per // 2
    _, n = B.shape
    A = A.astype(jnp.bfloat16)
    B = B.astype(jnp.bfloat16)

    def body(
        zt_ref, a_ref, b_ref, out_ref, comm_ref, cstage_ref,
        in_sem, copy_sems, zsend_sems, zrecv_sems,
        psend_p, precv_p, psend_m, precv_m,
    ):
        my = lax.axis_index("i")
        z = my // N_T
        t = my % N_T
        plane_r = N_T * z + (t + 1) % N_T
        plane_l = N_T * z + (t - 1) % N_T
        def ztab(j):
            return zt_ref[N_T * z + j]

        def col_mesh(zq):
            return N_T * zq + t

        barrier_sem = pltpu.get_barrier_semaphore()
        for nbr in (plane_l, plane_r):
            pl.semaphore_signal(
                barrier_sem, inc=1,
                device_id=(nbr,), device_id_type=pl.DeviceIdType.MESH,
            )
        for j in range(1, N_Z):
            pl.semaphore_signal(
                barrier_sem, inc=1,
                device_id=(col_mesh(ztab(j)),),
                device_id_type=pl.DeviceIdType.MESH,
            )
        pl.semaphore_wait(barrier_sem, 5)

        my_in = pltpu.make_async_copy(a_ref, comm_ref.at[my], in_sem)
        my_in.start()
        my_in.wait()

        def rdma(src, dev, ssem, rsem):
            return pltpu.make_async_remote_copy(
                src_ref=src, dst_ref=src,
                send_sem=ssem, recv_sem=rsem,
                device_id=(dev,), device_id_type=pl.DeviceIdType.MESH,
            )

        def full(o):
            return comm_ref.at[o]

        def half(o, which):
            return comm_ref.at[o, pl.ds(which * m_half, m_half)]

        idx = [0]

        def compute_and_store(origin):
            s = idx[0] % 2
            if idx[0] >= 2:
                pltpu.make_async_copy(
                    cstage_ref.at[s],
                    out_ref.at[pl.ds(origin * m_per, m_per)],
                    copy_sems.at[s],
                ).wait()
            cstage_ref[s] = jnp.dot(
                comm_ref[origin], b_ref[...],
                preferred_element_type=jnp.float32,
            ).astype(jnp.bfloat16)
            pltpu.make_async_copy(
                cstage_ref.at[s],
                out_ref.at[pl.ds(origin * m_per, m_per)],
                copy_sems.at[s],
            ).start()
            idx[0] += 1

        for j in range(1, N_Z):
            rdma(full(my), col_mesh(ztab(j)),
                 zsend_sems.at[j], zrecv_sems.at[z]).start()
        rdma(full(my), plane_r, psend_p.at[0], precv_p.at[0]).start()
        rdma(full(my), plane_l, psend_m.at[0], precv_m.at[0]).start()
        compute_and_store(my)

        for j in range(1, N_Z):
            zp = ztab(j)
            rdma(full(col_mesh(zp)), plane_l,
                 zsend_sems.at[0], zrecv_sems.at[zp]).wait_recv()
            rdma(full(col_mesh(zp)), plane_r,
                 psend_p.at[j], precv_p.at[j]).start()
            rdma(full(col_mesh(zp)), plane_l,
                 psend_m.at[j], precv_m.at[j]).start()
            compute_and_store(col_mesh(zp))

        for j in range(N_Z):
            zp = ztab(j)
            o_l = N_T * zp + (t - 1) % N_T
            o_r = N_T * zp + (t + 1) % N_T
            rdma(full(o_l), plane_l,
                 psend_p.at[j], precv_p.at[j]).wait_recv()
            rdma(half(o_l, 0), plane_r,
                 psend_p.at[N_Z + j], precv_p.at[N_Z + j]).start()
            compute_and_store(o_l)
            rdma(full(o_r), plane_r,
                 psend_m.at[j], precv_m.at[j]).wait_recv()
            rdma(half(o_r, 1), plane_l,
                 psend_m.at[N_Z + j], precv_m.at[N_Z + j]).start()
            compute_and_store(o_r)

        for j in range(N_Z):
            o_d = N_T * ztab(j) + (t + 2) % N_T
            rdma(half(o_d, 0), plane_l,
                 psend_p.at[N_Z + j], precv_p.at[N_Z + j]).wait_recv()
            rdma(half(o_d, 1), plane_r,
                 psend_m.at[N_Z + j], precv_m.at[N_Z + j]).wait_recv()
            compute_and_store(o_d)

        for j in range(1, N_Z):
            rdma(full(my), col_mesh(ztab(j)),
                 zsend_sems.at[j], zrecv_sems.at[z]).wait_send()
        for j in range(N_Z):
            rdma(full(my), plane_r, psend_p.at[j], precv_p.at[j]).wait_send()
            rdma(full(my), plane_l, psend_m.at[j], precv_m.at[j]).wait_send()
            rdma(half(my, 0), plane_r,
                 psend_p.at[N_Z + j], precv_p.at[N_Z + j]).wait_send()
            rdma(half(my, 1), plane_l,
                 psend_m.at[N_Z + j], precv_m.at[N_Z + j]).wait_send()
        for s in range(2):
            pltpu.make_async_copy(
                cstage_ref.at[s],
                out_ref.at[pl.ds(my * m_per, m_per)],
                copy_sems.at[s],
            ).wait()

    return pl.pallas_call(
        body,
        out_shape=jax.ShapeDtypeStruct((N_DEV * m_per, n), jnp.bfloat16),
        in_specs=[
            pl.BlockSpec(memory_space=pltpu.MemorySpace.SMEM),
            pl.BlockSpec(memory_space=pltpu.VMEM),
            pl.BlockSpec(memory_space=pltpu.VMEM),
        ],
        out_specs=pl.BlockSpec(memory_space=pl.ANY),
        scratch_shapes=[
            pltpu.VMEM((N_DEV, m_per, k), jnp.bfloat16),
            pltpu.VMEM((2, m_per, n), jnp.bfloat16),
            pltpu.SemaphoreType.DMA,
            pltpu.SemaphoreType.DMA((2,)),
            pltpu.SemaphoreType.DMA((N_Z,)),
            pltpu.SemaphoreType.DMA((N_Z,)),
            pltpu.SemaphoreType.DMA((2 * N_Z,)),
            pltpu.SemaphoreType.DMA((2 * N_Z,)),
            pltpu.SemaphoreType.DMA((2 * N_Z,)),
            pltpu.SemaphoreType.DMA((2 * N_Z,)),
        ],
        compiler_params=pltpu.CompilerParams(
            collective_id=0,
            vmem_limit_bytes=63 * 1024 * 1024,
        ),
    )(jnp.array(_ZTAB, dtype=jnp.int32), A, B)


# device time: 243185 ns/iter; 1.1282x vs baseline; 1.1282x over previous
import jax
import jax.numpy as jnp
from jax import lax
from jax.experimental import pallas as pl
from jax.experimental.pallas import tpu as pltpu

N_DEV = 16
N_Z = 4
N_T = 4


def kernel(A, B):
    m_per, k = A.shape
    m_half = m_per // 2
    _, n = B.shape

    def body(
        a_ref, b_ref, out_ref, comm_ref, cstage_ref, bbf_ref,
        in_sem, copy_sems, zsend_sems, zrecv_sems,
        psend_p, precv_p, psend_m, precv_m,
    ):
        my = lax.axis_index("i")
        z = my // N_T
        t = my % N_T
        plane_r = N_T * z + (t + 1) % N_T
        plane_l = N_T * z + (t - 1) % N_T

        def col_mesh(zq):
            return N_T * zq + t

        barrier_sem = pltpu.get_barrier_semaphore()
        for nbr in (plane_l, plane_r):
            pl.semaphore_signal(
                barrier_sem, inc=1,
                device_id=(nbr,), device_id_type=pl.DeviceIdType.MESH,
            )
        for zt in range(N_Z):
            @pl.when(zt != z)
            def _():
                pl.semaphore_signal(
                    barrier_sem, inc=1,
                    device_id=(col_mesh(zt),),
                    device_id_type=pl.DeviceIdType.MESH,
                )
        pl.semaphore_wait(barrier_sem, 5)

        comm_ref[my] = a_ref[...].astype(jnp.bfloat16)
        bbf_ref[...] = b_ref[...].astype(jnp.bfloat16)

        def rdma(src, dev, ssem, rsem):
            return pltpu.make_async_remote_copy(
                src_ref=src, dst_ref=src,
                send_sem=ssem, recv_sem=rsem,
                device_id=(dev,), device_id_type=pl.DeviceIdType.MESH,
            )

        def full(o):
            return comm_ref.at[o]

        def half(o, which):
            return comm_ref.at[o, pl.ds(which * m_half, m_half)]

        idx = [0]

        def compute_and_store(origin):
            s = idx[0] % 2
            if idx[0] >= 2:
                pltpu.make_async_copy(
                    cstage_ref.at[s],
                    out_ref.at[pl.ds(origin * m_per, m_per)],
                    copy_sems.at[s],
                ).wait()
            cstage_ref[s] = jnp.dot(
                comm_ref[origin], bbf_ref[...],
                preferred_element_type=jnp.float32,
            ).astype(jnp.bfloat16)
            pltpu.make_async_copy(
                cstage_ref.at[s],
                out_ref.at[pl.ds(origin * m_per, m_per)],
                copy_sems.at[s],
            ).start()
            idx[0] += 1

        for zt in range(N_Z):
            @pl.when(zt != z)
            def _():
                rdma(full(my), col_mesh(zt),
                     zsend_sems.at[zt], zrecv_sems.at[z]).start()
        rdma(full(my), plane_r, psend_p.at[z], precv_p.at[z]).start()
        rdma(full(my), plane_l, psend_m.at[z], precv_m.at[z]).start()
        compute_and_store(my)

        for d in range(1, N_Z):
            for sgn in (-1, 1):
                z2 = z + sgn * d
                valid = jnp.logical_and(z2 >= 0, z2 <= N_Z - 1)
                z2c = jnp.clip(z2, 0, N_Z - 1)

                @pl.when(valid)
                def _():
                    rdma(full(col_mesh(z2c)), plane_l,
                         zsend_sems.at[0], zrecv_sems.at[z2c]).wait_recv()
                    rdma(full(col_mesh(z2c)), plane_r,
                         psend_p.at[z2c], precv_p.at[z2c]).start()
                    rdma(full(col_mesh(z2c)), plane_l,
                         psend_m.at[z2c], precv_m.at[z2c]).start()
                compute_and_store(col_mesh(z2c))

        for zq in range(N_Z):
            o_l = N_T * zq + (t - 1) % N_T
            o_r = N_T * zq + (t + 1) % N_T
            rdma(full(o_l), plane_l,
                 psend_p.at[zq], precv_p.at[zq]).wait_recv()
            rdma(half(o_l, 0), plane_r,
                 psend_p.at[N_Z + zq], precv_p.at[N_Z + zq]).start()
            compute_and_store(o_l)
            rdma(full(o_r), plane_r,
                 psend_m.at[zq], precv_m.at[zq]).wait_recv()
            rdma(half(o_r, 1), plane_l,
                 psend_m.at[N_Z + zq], precv_m.at[N_Z + zq]).start()
            compute_and_store(o_r)

        for zq in range(N_Z):
            o_d = N_T * zq + (t + 2) % N_T
            rdma(half(o_d, 0), plane_l,
                 psend_p.at[N_Z + zq], precv_p.at[N_Z + zq]).wait_recv()
            rdma(half(o_d, 1), plane_r,
                 psend_m.at[N_Z + zq], precv_m.at[N_Z + zq]).wait_recv()
            compute_and_store(o_d)

        for zt in range(N_Z):
            @pl.when(zt != z)
            def _():
                rdma(full(my), col_mesh(zt),
                     zsend_sems.at[zt], zrecv_sems.at[z]).wait_send()
        for i in range(N_Z):
            rdma(full(my), plane_r, psend_p.at[i], precv_p.at[i]).wait_send()
            rdma(full(my), plane_l, psend_m.at[i], precv_m.at[i]).wait_send()
        for i in range(N_Z, 2 * N_Z):
            rdma(half(my, 0), plane_r, psend_p.at[i], precv_p.at[i]).wait_send()
            rdma(half(my, 1), plane_l, psend_m.at[i], precv_m.at[i]).wait_send()
        for s in range(2):
            pltpu.make_async_copy(
                cstage_ref.at[s],
                out_ref.at[pl.ds(my * m_per, m_per)],
                copy_sems.at[s],
            ).wait()

    return pl.pallas_call(
        body,
        out_shape=jax.ShapeDtypeStruct((N_DEV * m_per, n), jnp.bfloat16),
        in_specs=[
            pl.BlockSpec(memory_space=pltpu.VMEM),
            pl.BlockSpec(memory_space=pltpu.VMEM),
        ],
        out_specs=pl.BlockSpec(memory_space=pl.ANY),
        scratch_shapes=[
            pltpu.VMEM((N_DEV, m_per, k), jnp.bfloat16),
            pltpu.VMEM((2, m_per, n), jnp.bfloat16),
            pltpu.VMEM((k, n), jnp.bfloat16),
            pltpu.SemaphoreType.DMA,
            pltpu.SemaphoreType.DMA((2,)),
            pltpu.SemaphoreType.DMA((N_Z,)),
            pltpu.SemaphoreType.DMA((N_Z,)),
            pltpu.SemaphoreType.DMA((2 * N_Z,)),
            pltpu.SemaphoreType.DMA((2 * N_Z,)),
            pltpu.SemaphoreType.DMA((2 * N_Z,)),
            pltpu.SemaphoreType.DMA((2 * N_Z,)),
        ],
        compiler_params=pltpu.CompilerParams(
            collective_id=0,
            vmem_limit_bytes=63 * 1024 * 1024,
        ),
    )(A, B)
